# Initial kernel scaffold; baseline (speedup 1.0000x reference)
import jax
import jax.numpy as jnp
from jax.experimental import pallas as pl


def kernel(q, mask, codes, top_m):
    raise NotImplementedError("write your pallas kernel here")


# speedup vs baseline: 56.9673x; 56.9673x over previous
import functools

import jax
import jax.numpy as jnp
from jax.experimental import pallas as pl


def _fused_body(q_ref, mask_ref, codes_ref, sim_ref, idx_ref, w_ref, *, m, k):
    tile = jax.lax.dot_general(
        q_ref[...], codes_ref[...],
        dimension_numbers=(((1,), (1,)), ((), ())),
        preferred_element_type=jnp.float32,
    )
    sim_ref[...] = tile

    bn = tile.shape[0]
    gidx = jax.lax.broadcasted_iota(jnp.int32, tile.shape, 1)
    work = tile
    vals = []
    idxs = []
    for t in range(m):
        mx = jnp.max(work, axis=1, keepdims=True)
        gi = jnp.min(jnp.where(work == mx, gidx, k), axis=1, keepdims=True)
        vals.append(mx)
        idxs.append(gi)
        if t + 1 < m:
            work = jnp.where(gidx == gi, -jnp.inf, work)

    v = jnp.concatenate(vals, axis=1)
    ii = jnp.concatenate(idxs, axis=1)

    e = jnp.exp(v - v[:, :1])
    w = e / jnp.sum(e, axis=1, keepdims=True)

    mrow = mask_ref[...]
    w = w * mrow
    iota_m = jax.lax.broadcasted_iota(jnp.int32, (bn, m), 1)
    ii = jnp.where(mrow == 0.0, iota_m, ii)

    idx_ref[...] = ii
    w_ref[...] = w


def _run(q, mask, codes, top_m):
    B, N, D = q.shape
    K = codes.shape[0]
    BN = B * N
    M = 4

    bn = 128
    while BN % bn:
        bn //= 2

    q2 = q.reshape(BN, D)
    mask2 = mask.reshape(BN, 1)

    grid = (BN // bn,)
    sim, idx, w = pl.pallas_call(
        functools.partial(_fused_body, m=M, k=K),
        grid=grid,
        in_specs=[
            pl.BlockSpec((bn, D), lambda i: (i, 0)),
            pl.BlockSpec((bn, 1), lambda i: (i, 0)),
            pl.BlockSpec((K, D), lambda i: (0, 0)),
        ],
        out_specs=[
            pl.BlockSpec((bn, K), lambda i: (i, 0)),
            pl.BlockSpec((bn, M), lambda i: (i, 0)),
            pl.BlockSpec((bn, M), lambda i: (i, 0)),
        ],
        out_shape=[
            jax.ShapeDtypeStruct((BN, K), jnp.float32),
            jax.ShapeDtypeStruct((BN, M), jnp.int32),
            jax.ShapeDtypeStruct((BN, M), jnp.float32),
        ],
    )(q2, mask2, codes)

    weights = w + (jnp.asarray(top_m) * 0).astype(w.dtype)
    return idx.reshape(B, N, M), weights.reshape(B, N, M), sim.reshape(B, N, K)


def kernel(q, mask, codes, top_m):
    return _run(q, mask, codes, top_m)


# speedup vs baseline: 62.2636x; 1.0930x over previous
import functools

import jax
import jax.numpy as jnp
from jax.experimental import pallas as pl


def _fused_body(q_ref, mask_ref, codes_ref, sim_ref, idx_ref, w_ref, *, m, k):
    tile = jax.lax.dot_general(
        q_ref[...], codes_ref[...],
        dimension_numbers=(((1,), (1,)), ((), ())),
        preferred_element_type=jnp.float32,
    )
    sim_ref[...] = tile

    bn = tile.shape[0]
    gidx = jax.lax.broadcasted_iota(jnp.int32, tile.shape, 1)
    work = tile
    vals = []
    idxs = []
    for t in range(m):
        mx = jnp.max(work, axis=1, keepdims=True)
        gi = jnp.min(jnp.where(work == mx, gidx, k), axis=1, keepdims=True)
        vals.append(mx)
        idxs.append(gi)
        if t + 1 < m:
            work = jnp.where(gidx == gi, -jnp.inf, work)

    v = jnp.concatenate(vals, axis=1)
    ii = jnp.concatenate(idxs, axis=1)

    e = jnp.exp(v - v[:, :1])
    w = e / jnp.sum(e, axis=1, keepdims=True)

    mrow = mask_ref[...]
    w = w * mrow
    iota_m = jax.lax.broadcasted_iota(jnp.int32, (bn, m), 1)
    ii = jnp.where(mrow == 0.0, iota_m, ii)

    idx_ref[...] = ii
    w_ref[...] = w


def _run(q, mask, codes, top_m):
    B, N, D = q.shape
    K = codes.shape[0]
    BN = B * N
    M = 4

    bn = 256
    while BN % bn:
        bn //= 2

    q2 = q.reshape(BN, D)
    mask2 = mask.reshape(BN, 1)

    grid = (BN // bn,)
    sim, idx, w = pl.pallas_call(
        functools.partial(_fused_body, m=M, k=K),
        grid=grid,
        in_specs=[
            pl.BlockSpec((bn, D), lambda i: (i, 0)),
            pl.BlockSpec((bn, 1), lambda i: (i, 0)),
            pl.BlockSpec((K, D), lambda i: (0, 0)),
        ],
        out_specs=[
            pl.BlockSpec((bn, K), lambda i: (i, 0)),
            pl.BlockSpec((bn, M), lambda i: (i, 0)),
            pl.BlockSpec((bn, M), lambda i: (i, 0)),
        ],
        out_shape=[
            jax.ShapeDtypeStruct((BN, K), jnp.float32),
            jax.ShapeDtypeStruct((BN, M), jnp.int32),
            jax.ShapeDtypeStruct((BN, M), jnp.float32),
        ],
    )(q2, mask2, codes)

    weights = w + (jnp.asarray(top_m) * 0).astype(w.dtype)
    return idx.reshape(B, N, M), weights.reshape(B, N, M), sim.reshape(B, N, K)


def kernel(q, mask, codes, top_m):
    return _run(q, mask, codes, top_m)


# speedup vs baseline: 66.1806x; 1.0629x over previous
import functools

import jax
import jax.numpy as jnp
from jax.experimental import pallas as pl


def _fused_body(q_ref, mask_ref, codes_ref, sim_ref, idx_ref, w_ref, *, m, k):
    tile = jax.lax.dot_general(
        q_ref[...], codes_ref[...],
        dimension_numbers=(((1,), (1,)), ((), ())),
        preferred_element_type=jnp.float32,
    )
    sim_ref[...] = tile

    bn = tile.shape[0]
    gidx = jax.lax.broadcasted_iota(jnp.int32, tile.shape, 1)
    work = tile
    vals = []
    idxs = []
    for t in range(m):
        mx = jnp.max(work, axis=1, keepdims=True)
        gi = jnp.min(jnp.where(work == mx, gidx, k), axis=1, keepdims=True)
        vals.append(mx)
        idxs.append(gi)
        if t + 1 < m:
            work = jnp.where(gidx == gi, -jnp.inf, work)

    v = jnp.concatenate(vals, axis=1)
    ii = jnp.concatenate(idxs, axis=1)

    e = jnp.exp(v - v[:, :1])
    w = e / jnp.sum(e, axis=1, keepdims=True)

    mrow = mask_ref[...]
    w = w * mrow
    iota_m = jax.lax.broadcasted_iota(jnp.int32, (bn, m), 1)
    ii = jnp.where(mrow == 0.0, iota_m, ii)

    idx_ref[...] = ii
    w_ref[...] = w


def _run(q, mask, codes, top_m):
    B, N, D = q.shape
    K = codes.shape[0]
    BN = B * N
    M = 4

    bn = 512
    while BN % bn:
        bn //= 2

    q2 = q.reshape(BN, D)
    mask2 = mask.reshape(BN, 1)

    grid = (BN // bn,)
    sim, idx, w = pl.pallas_call(
        functools.partial(_fused_body, m=M, k=K),
        grid=grid,
        in_specs=[
            pl.BlockSpec((bn, D), lambda i: (i, 0)),
            pl.BlockSpec((bn, 1), lambda i: (i, 0)),
            pl.BlockSpec((K, D), lambda i: (0, 0)),
        ],
        out_specs=[
            pl.BlockSpec((bn, K), lambda i: (i, 0)),
            pl.BlockSpec((bn, M), lambda i: (i, 0)),
            pl.BlockSpec((bn, M), lambda i: (i, 0)),
        ],
        out_shape=[
            jax.ShapeDtypeStruct((BN, K), jnp.float32),
            jax.ShapeDtypeStruct((BN, M), jnp.int32),
            jax.ShapeDtypeStruct((BN, M), jnp.float32),
        ],
    )(q2, mask2, codes)

    weights = w + (jnp.asarray(top_m) * 0).astype(w.dtype)
    return idx.reshape(B, N, M), weights.reshape(B, N, M), sim.reshape(B, N, K)


def kernel(q, mask, codes, top_m):
    return _run(q, mask, codes, top_m)


# speedup vs baseline: 66.2377x; 1.0009x over previous
import functools

import jax
import jax.numpy as jnp
from jax.experimental import pallas as pl
from jax.experimental.pallas import tpu as pltpu


def _fused_body(q_ref, mask_ref, codes_ref, sim_ref, idx_ref, w_ref, *, m, k):
    tile = jax.lax.dot_general(
        q_ref[...], codes_ref[...],
        dimension_numbers=(((1,), (1,)), ((), ())),
        preferred_element_type=jnp.float32,
    )
    sim_ref[...] = tile

    bn = tile.shape[0]
    gidx = jax.lax.broadcasted_iota(jnp.int32, tile.shape, 1)
    work = tile
    vals = []
    idxs = []
    for t in range(m):
        mx = jnp.max(work, axis=1, keepdims=True)
        gi = jnp.min(jnp.where(work == mx, gidx, k), axis=1, keepdims=True)
        vals.append(mx)
        idxs.append(gi)
        if t + 1 < m:
            work = jnp.where(gidx == gi, -jnp.inf, work)

    v = jnp.concatenate(vals, axis=1)
    ii = jnp.concatenate(idxs, axis=1)

    e = jnp.exp(v - v[:, :1])
    w = e / jnp.sum(e, axis=1, keepdims=True)

    mrow = mask_ref[...]
    w = w * mrow
    iota_m = jax.lax.broadcasted_iota(jnp.int32, (bn, m), 1)
    ii = jnp.where(mrow == 0.0, iota_m, ii)

    idx_ref[...] = ii
    w_ref[...] = w


def _run(q, mask, codes, top_m):
    B, N, D = q.shape
    K = codes.shape[0]
    BN = B * N
    M = 4

    bn = 512
    while BN % bn:
        bn //= 2

    q2 = q.reshape(BN, D)
    mask2 = mask.reshape(BN, 1)

    grid = (BN // bn,)
    sim, idx, w = pl.pallas_call(
        functools.partial(_fused_body, m=M, k=K),
        grid=grid,
        in_specs=[
            pl.BlockSpec((bn, D), lambda i: (i, 0)),
            pl.BlockSpec((bn, 1), lambda i: (i, 0)),
            pl.BlockSpec((K, D), lambda i: (0, 0)),
        ],
        out_specs=[
            pl.BlockSpec((bn, K), lambda i: (i, 0)),
            pl.BlockSpec((bn, M), lambda i: (i, 0)),
            pl.BlockSpec((bn, M), lambda i: (i, 0)),
        ],
        out_shape=[
            jax.ShapeDtypeStruct((BN, K), jnp.float32),
            jax.ShapeDtypeStruct((BN, M), jnp.int32),
            jax.ShapeDtypeStruct((BN, M), jnp.float32),
        ],
        compiler_params=pltpu.CompilerParams(
            dimension_semantics=("parallel",),
        ),
    )(q2, mask2, codes)

    weights = w + (jnp.asarray(top_m) * 0).astype(w.dtype)
    return idx.reshape(B, N, M), weights.reshape(B, N, M), sim.reshape(B, N, K)


def kernel(q, mask, codes, top_m):
    return _run(q, mask, codes, top_m)


# speedup vs baseline: 72.9093x; 1.1007x over previous
import functools

import jax
import jax.numpy as jnp
from jax.experimental import pallas as pl
from jax.experimental.pallas import tpu as pltpu


def _fused_body(q_ref, mask_ref, codes_ref, sim_ref, idx_ref, w_ref, *, m, k):
    tile = jax.lax.dot_general(
        q_ref[...], codes_ref[...],
        dimension_numbers=(((1,), (1,)), ((), ())),
        preferred_element_type=jnp.float32,
    )
    sim_ref[...] = tile

    bn = tile.shape[0]
    niota = (-jax.lax.broadcasted_iota(jnp.int32, tile.shape, 1)).astype(jnp.float32)
    work = tile
    vals = []
    nidxs = []
    for t in range(m):
        mx = jnp.max(work, axis=1, keepdims=True)
        cand = jnp.where(work == mx, niota, jnp.float32(-3e38))
        gi = jnp.max(cand, axis=1, keepdims=True)
        vals.append(mx)
        nidxs.append(gi)
        if t + 1 < m:
            work = jnp.where(cand == gi, -jnp.inf, work)

    v = jnp.concatenate(vals, axis=1)
    ii = (-jnp.concatenate(nidxs, axis=1)).astype(jnp.int32)

    e = jnp.exp(v - v[:, :1])
    w = e / jnp.sum(e, axis=1, keepdims=True)

    mrow = mask_ref[...]
    w = w * mrow
    iota_m = jax.lax.broadcasted_iota(jnp.int32, (bn, m), 1)
    ii = jnp.where(mrow == 0.0, iota_m, ii)

    idx_ref[...] = ii
    w_ref[...] = w


def _run(q, mask, codes, top_m):
    B, N, D = q.shape
    K = codes.shape[0]
    BN = B * N
    M = 4

    bn = 256
    while BN % bn:
        bn //= 2

    q2 = q.reshape(BN, D)
    mask2 = mask.reshape(BN, 1)

    grid = (BN // bn,)
    sim, idx, w = pl.pallas_call(
        functools.partial(_fused_body, m=M, k=K),
        grid=grid,
        in_specs=[
            pl.BlockSpec((bn, D), lambda i: (i, 0)),
            pl.BlockSpec((bn, 1), lambda i: (i, 0)),
            pl.BlockSpec((K, D), lambda i: (0, 0)),
        ],
        out_specs=[
            pl.BlockSpec((bn, K), lambda i: (i, 0)),
            pl.BlockSpec((bn, M), lambda i: (i, 0)),
            pl.BlockSpec((bn, M), lambda i: (i, 0)),
        ],
        out_shape=[
            jax.ShapeDtypeStruct((BN, K), jnp.float32),
            jax.ShapeDtypeStruct((BN, M), jnp.int32),
            jax.ShapeDtypeStruct((BN, M), jnp.float32),
        ],
        compiler_params=pltpu.CompilerParams(
            dimension_semantics=("parallel",),
        ),
    )(q2, mask2, codes)

    weights = w + (jnp.asarray(top_m) * 0).astype(w.dtype)
    return idx.reshape(B, N, M), weights.reshape(B, N, M), sim.reshape(B, N, K)


def kernel(q, mask, codes, top_m):
    return _run(q, mask, codes, top_m)
